# Initial kernel scaffold; baseline (speedup 1.0000x reference)
import jax
import jax.numpy as jnp
from jax.experimental import pallas as pl


def kernel(x, pos, edge_index, W_embed, W_msg, W_upd, W_out):
    raise NotImplementedError("write your pallas kernel here")


# speedup vs baseline: 1.8296x; 1.8296x over previous
import functools

import jax
import jax.numpy as jnp
from jax import lax
from jax.experimental import pallas as pl
from jax.experimental.pallas import tpu as pltpu
from jax.experimental.pallas import tpu_sc as plsc

N = 50000
E = 800000
D_IN = 4
HID = 64
H2 = 32
NB = 10
SH_DIM = 9
NF = SH_DIM + NB
LAYERS = 3
MAX_R = 2.0
INV_NEIGH = 1.0 / 16.0

NC, NS = 2, 16
NW = NC * NS
CHUNK = 128
EP = NW * 196 * CHUNK
NCHUNK = EP // CHUNK
EB = 2048
NBLK = 2000
STRIPE = N // NS
ZROWS = 625
NEG = -1e30

_mesh = functools.partial(
    plsc.VectorSubcoreMesh,
    core_axis_name="c", subcore_axis_name="s", num_cores=NC, num_subcores=NS,
)
_SC_PARAMS = pltpu.CompilerParams(use_tc_tiling_on_sc=False)


def _embed_body(x_ref, we_ref, wh_ref, h_ref, hw_ref):
    h = jnp.maximum(jnp.dot(x_ref[...], we_ref[...],
                            preferred_element_type=jnp.float32), 0.0)
    h_ref[...] = h
    hw = jnp.dot(h, wh_ref[...], preferred_element_type=jnp.float32)
    hw_ref[0] = hw[:, :H2]
    hw_ref[1] = hw[:, H2:]


def _stage_embed(x, W_embed, Wh0):
    return pl.pallas_call(
        _embed_body,
        grid=(N // NBLK,),
        in_specs=[
            pl.BlockSpec((NBLK, D_IN), lambda i: (i, 0)),
            pl.BlockSpec((D_IN, HID), lambda i: (0, 0)),
            pl.BlockSpec((HID, HID), lambda i: (0, 0)),
        ],
        out_specs=[
            pl.BlockSpec((NBLK, HID), lambda i: (i, 0)),
            pl.BlockSpec((NC, NBLK, H2), lambda i: (0, i, 0)),
        ],
        out_shape=[
            jax.ShapeDtypeStruct((N, HID), jnp.float32),
            jax.ShapeDtypeStruct((NC, N, H2), jnp.float32),
        ],
    )(x, W_embed, Wh0)


def _stage_rel(posp, srcp, dstp):
    @functools.partial(
        pl.kernel,
        mesh=_mesh(),
        out_type=jax.ShapeDtypeStruct((EP, 16), jnp.float32),
        scratch_types=[
            pltpu.VMEM((CHUNK,), jnp.int32),
            pltpu.VMEM((CHUNK,), jnp.int32),
            pltpu.VMEM((CHUNK, 16), jnp.float32),
            pltpu.VMEM((CHUNK, 16), jnp.float32),
            pltpu.SemaphoreType.DMA,
            pltpu.SemaphoreType.DMA,
        ],
        compiler_params=_SC_PARAMS,
    )
    def k(posp_hbm, src_hbm, dst_hbm, rel_hbm, sidx, didx, ps, pd, sem1, sem2):
        w = lax.axis_index("s") * NC + lax.axis_index("c")
        per = NCHUNK // NW

        @pl.loop(0, per)
        def _chunk(j):
            base = (w * per + j) * CHUNK
            pltpu.sync_copy(src_hbm.at[pl.ds(base, CHUNK)], sidx)
            pltpu.sync_copy(dst_hbm.at[pl.ds(base, CHUNK)], didx)
            cs = pltpu.async_copy(posp_hbm.at[sidx], ps, sem1)
            cd = pltpu.async_copy(posp_hbm.at[didx], pd, sem2)
            cs.wait()
            cd.wait()

            @plsc.parallel_loop(0, CHUNK, unroll=8)
            def _row(r):
                pd[r, :] = pd[r, :] - ps[r, :]

            pltpu.sync_copy(pd, rel_hbm.at[pl.ds(base, CHUNK)])

    return k(posp, srcp, dstp)


def _bias_body(rel_ref, w_ref, out_ref):
    i = pl.program_id(0)
    rx = rel_ref[:, 0]
    ry = rel_ref[:, 1]
    rz = rel_ref[:, 2]
    r = jnp.sqrt(rx * rx + ry * ry + rz * rz + 1e-12)
    inv = 1.0 / (r + 1e-8)
    ux = rx * inv
    uy = ry * inv
    uz = rz * inv
    c3 = 3.0 ** 0.5
    c5 = 5.0 ** 0.5
    c15 = 15.0 ** 0.5
    feats = [
        jnp.ones_like(ux), c3 * ux, c3 * uy, c3 * uz,
        c15 * ux * uy, c15 * uy * uz,
        c5 * 0.5 * (2.0 * uz * uz - ux * ux - uy * uy),
        c15 * ux * uz, c15 * 0.5 * (ux * ux - uy * uy),
    ]
    width = MAX_R / NB
    for kk in range(NB):
        ck = MAX_R * kk / (NB - 1)
        feats.append(jnp.exp(-(((r - ck) / width) ** 2)))
    featT = jnp.stack(feats, axis=0)
    y = lax.dot_general(featT, w_ref[...], (((0,), (0,)), ((), ())),
                        preferred_element_type=jnp.float32)
    eidx = i * EB + lax.broadcasted_iota(jnp.int32, (EB, 1), 0)
    valid = eidx < E
    for c in range(NC):
        for l in range(LAYERS):
            blk = y[:, l * HID + c * H2: l * HID + (c + 1) * H2]
            out_ref[c, l] = jnp.where(valid, blk, NEG)


def _stage_bias(relp, Wcat):
    return pl.pallas_call(
        _bias_body,
        grid=(EP // EB,),
        in_specs=[
            pl.BlockSpec((EB, 16), lambda i: (i, 0)),
            pl.BlockSpec((NF, 3 * HID), lambda i: (0, 0)),
        ],
        out_specs=pl.BlockSpec((NC, LAYERS, EB, H2), lambda i: (0, 0, i, 0)),
        out_shape=jax.ShapeDtypeStruct((NC, LAYERS, EP, H2), jnp.float32),
    )(relp, Wcat)


def _stage_edges(l, hw_flat, bias_flat, srcp, dstp):
    @functools.partial(
        pl.kernel,
        mesh=_mesh(),
        out_type=jax.ShapeDtypeStruct((NC * N, H2), jnp.float32),
        scratch_types=[
            pltpu.VMEM((CHUNK,), jnp.int32),
            pltpu.VMEM((CHUNK,), jnp.int32),
            pltpu.VMEM((CHUNK,), jnp.int32),
            pltpu.VMEM((CHUNK, H2), jnp.float32),
            pltpu.VMEM((CHUNK, H2), jnp.float32),
            pltpu.VMEM((ZROWS, H2), jnp.float32),
            pltpu.VMEM_SHARED((N, H2), jnp.float32),
            pltpu.SemaphoreType.DMA,
            pltpu.SemaphoreType.DMA,
        ],
        compiler_params=_SC_PARAMS,
    )
    def k(hw_hbm, bias_hbm, src_hbm, dst_hbm, agg_hbm,
          sidx, gidx, didx, grows, bbuf, zbuf, aggsh, sem1, sem2):
        c = lax.axis_index("c")
        s = lax.axis_index("s")

        @plsc.parallel_loop(0, ZROWS, unroll=8)
        def _z(rr):
            zbuf[rr, pl.ds(0, 16)] = jnp.zeros((16,), jnp.float32)
            zbuf[rr, pl.ds(16, 16)] = jnp.zeros((16,), jnp.float32)

        for kk in range(STRIPE // ZROWS):
            pltpu.sync_copy(zbuf, aggsh.at[pl.ds(s * STRIPE + kk * ZROWS, ZROWS)])
        plsc.subcore_barrier()

        per = NCHUNK // NS
        cN = c * N
        boff = (c * LAYERS + l) * EP

        @pl.loop(0, per)
        def _chunk(j):
            base = (s * per + j) * CHUNK
            pltpu.sync_copy(src_hbm.at[pl.ds(base, CHUNK)], sidx)
            pltpu.sync_copy(dst_hbm.at[pl.ds(base, CHUNK)], didx)

            @plsc.parallel_loop(0, CHUNK // 16, unroll=8)
            def _adj(g):
                gidx[pl.ds(g * 16, 16)] = sidx[pl.ds(g * 16, 16)] + cN

            cg = pltpu.async_copy(hw_hbm.at[gidx], grows, sem1)
            cb = pltpu.async_copy(bias_hbm.at[pl.ds(boff + base, CHUNK)], bbuf, sem2)
            cg.wait()
            cb.wait()

            @plsc.parallel_loop(0, CHUNK, unroll=4)
            def _row(r):
                g0 = grows[r, pl.ds(0, 16)] + bbuf[r, pl.ds(0, 16)]
                g1 = grows[r, pl.ds(16, 16)] + bbuf[r, pl.ds(16, 16)]
                grows[r, pl.ds(0, 16)] = jnp.maximum(g0, 0.0)
                grows[r, pl.ds(16, 16)] = jnp.maximum(g1, 0.0)

            pltpu.sync_copy(grows, aggsh.at[didx], add=True)

        plsc.subcore_barrier()
        for kk in range(STRIPE // ZROWS):
            off = s * STRIPE + kk * ZROWS
            pltpu.sync_copy(aggsh.at[pl.ds(off, ZROWS)],
                            agg_hbm.at[pl.ds(cN + off, ZROWS)])

    return k(hw_flat, bias_flat, srcp, dstp)


def _upd_body(h_ref, agg_ref, wt_ref, wb_ref, wh_ref, h_out, hw_out):
    z = jnp.dot(h_ref[...], wt_ref[...], preferred_element_type=jnp.float32)
    z += (jnp.dot(agg_ref[0], wb_ref[0], preferred_element_type=jnp.float32)
          + jnp.dot(agg_ref[1], wb_ref[1], preferred_element_type=jnp.float32)
          ) * INV_NEIGH
    hn = jnp.maximum(z, 0.0)
    h_out[...] = hn
    hw = jnp.dot(hn, wh_ref[...], preferred_element_type=jnp.float32)
    hw_out[0] = hw[:, :H2]
    hw_out[1] = hw[:, H2:]


def _stage_update(h, agg, wt, wb, wh_next):
    return pl.pallas_call(
        _upd_body,
        grid=(N // NBLK,),
        in_specs=[
            pl.BlockSpec((NBLK, HID), lambda i: (i, 0)),
            pl.BlockSpec((NC, NBLK, H2), lambda i: (0, i, 0)),
            pl.BlockSpec((HID, HID), lambda i: (0, 0)),
            pl.BlockSpec((NC, H2, HID), lambda i: (0, 0, 0)),
            pl.BlockSpec((HID, HID), lambda i: (0, 0)),
        ],
        out_specs=[
            pl.BlockSpec((NBLK, HID), lambda i: (i, 0)),
            pl.BlockSpec((NC, NBLK, H2), lambda i: (0, i, 0)),
        ],
        out_shape=[
            jax.ShapeDtypeStruct((N, HID), jnp.float32),
            jax.ShapeDtypeStruct((NC, N, H2), jnp.float32),
        ],
    )(h, agg, wt, wb, wh_next)


def _final_body(h_ref, agg_ref, wt_ref, wb_ref, wo_ref, acc_ref, out_ref):
    i = pl.program_id(0)
    z = jnp.dot(h_ref[...], wt_ref[...], preferred_element_type=jnp.float32)
    z += (jnp.dot(agg_ref[0], wb_ref[0], preferred_element_type=jnp.float32)
          + jnp.dot(agg_ref[1], wb_ref[1], preferred_element_type=jnp.float32)
          ) * INV_NEIGH
    hn = jnp.maximum(z, 0.0)

    @pl.when(i == 0)
    def _():
        acc_ref[...] = jnp.zeros_like(acc_ref)

    acc_ref[...] += jnp.sum(hn, axis=0, keepdims=True)

    @pl.when(i == pl.num_programs(0) - 1)
    def _():
        out_ref[...] = jnp.dot(acc_ref[...], wo_ref[...],
                               preferred_element_type=jnp.float32) * (1.0 / N)


def _stage_final(h, agg, wt, wb, W_out):
    _, out = pl.pallas_call(
        _final_body,
        grid=(N // NBLK,),
        in_specs=[
            pl.BlockSpec((NBLK, HID), lambda i: (i, 0)),
            pl.BlockSpec((NC, NBLK, H2), lambda i: (0, i, 0)),
            pl.BlockSpec((HID, HID), lambda i: (0, 0)),
            pl.BlockSpec((NC, H2, HID), lambda i: (0, 0, 0)),
            pl.BlockSpec((HID, 3), lambda i: (0, 0)),
        ],
        out_specs=[
            pl.BlockSpec((1, HID), lambda i: (0, 0)),
            pl.BlockSpec((1, 3), lambda i: (0, 0)),
        ],
        out_shape=[
            jax.ShapeDtypeStruct((1, HID), jnp.float32),
            jax.ShapeDtypeStruct((1, 3), jnp.float32),
        ],
    )(h, agg, wt, wb, W_out)
    return out


def kernel(x, pos, edge_index, W_embed, W_msg, W_upd, W_out):
    src = edge_index[0].astype(jnp.int32)
    dst = edge_index[1].astype(jnp.int32)
    pad = EP - E
    srcp = jnp.concatenate([src, jnp.zeros((pad,), jnp.int32)])
    dstp = jnp.concatenate([dst, jnp.zeros((pad,), jnp.int32)])
    posp = jnp.zeros((N, 16), jnp.float32).at[:, :3].set(pos)
    Wcat = jnp.concatenate([W_msg[0][HID:], W_msg[1][HID:], W_msg[2][HID:]],
                           axis=1)

    h, hw = _stage_embed(x, W_embed, W_msg[0][:HID])
    relp = _stage_rel(posp, srcp, dstp)
    bias = _stage_bias(relp, Wcat)
    bias_flat = bias.reshape(NC * LAYERS * EP, H2)

    out = None
    for l in range(LAYERS):
        hw_flat = hw.reshape(NC * N, H2)
        agg = _stage_edges(l, hw_flat, bias_flat, srcp, dstp).reshape(NC, N, H2)
        wt = W_upd[l][:HID]
        wb = jnp.stack([W_upd[l][HID:HID + H2], W_upd[l][HID + H2:]])
        if l < LAYERS - 1:
            h, hw = _stage_update(h, agg, wt, wb, W_msg[l + 1][:HID])
        else:
            out = _stage_final(h, agg, wt, wb, W_out)
    return out[0, :2]
